# baseline (speedup 1.0000x reference)
import functools

import jax
import jax.numpy as jnp
from jax import lax
from jax.experimental import pallas as pl
from jax.experimental.pallas import tpu as pltpu
from jax.experimental.pallas import tpu_sc as plsc

N = 10000
NPAD = 10240
E = 320000
D = 128
NW = 32
EPW = E // NW
C = 100
NCHUNK = EPW // C
S = 25
NSB = NCHUNK // S
RPS = NPAD // 16
TB = 1000


def _sc_agg(with_deg):
    out_type = [jax.ShapeDtypeStruct((2, NPAD, D), jnp.float32)]
    if with_deg:
        out_type.append(jax.ShapeDtypeStruct((2, NPAD), jnp.float32))

    mesh = plsc.VectorSubcoreMesh(core_axis_name="c", subcore_axis_name="s")
    scratch = [
        pltpu.VMEM_SHARED((NPAD, D), jnp.float32),
        pltpu.VMEM_SHARED((NPAD,), jnp.float32),
        pltpu.VMEM((S, C), jnp.int32),
        pltpu.VMEM((S, C), jnp.int32),
        pltpu.VMEM((S, C), jnp.int32),
        pltpu.VMEM((S, C), jnp.int32),
        pltpu.VMEM((C, D), jnp.float32),
        pltpu.VMEM((C, D), jnp.float32),
        pltpu.VMEM((C,), jnp.float32),
        pltpu.SemaphoreType.DMA,
        pltpu.SemaphoreType.DMA,
        pltpu.SemaphoreType.DMA,
        pltpu.SemaphoreType.DMA,
    ]

    @functools.partial(pl.kernel, out_type=out_type, mesh=mesh,
                       scratch_types=scratch)
    def body(h_hbm, src_hbm, dst_hbm, zero_hbm, zerod_hbm, ones_hbm, *rest):
        if with_deg:
            agg_out, deg_out = rest[0], rest[1]
            rest = rest[2:]
        else:
            agg_out = rest[0]
            rest = rest[1:]
        (agg_sh, deg_sh, src0, src1, dst0, dst1, rows0, rows1, ones_v,
         sem0, sem1, isem, zsem) = rest

        c = lax.axis_index("c")
        s = lax.axis_index("s")
        wid = c * 16 + s

        srcb = (src0, src1)
        dstb = (dst0, dst1)
        rows = (rows0, rows1)
        sems = (sem0, sem1)

        def stage_idx(sb):
            bb = sb % 2
            return (pltpu.async_copy(src_hbm.at[wid, sb], srcb[bb], isem),
                    pltpu.async_copy(dst_hbm.at[wid, sb], dstb[bb], isem))

        idescs = {0: stage_idx(0)}
        row0 = s * RPS
        zd = [pltpu.async_copy(zero_hbm, agg_sh.at[pl.ds(row0, RPS)], zsem)]
        if with_deg:
            zd.append(pltpu.async_copy(zerod_hbm, deg_sh.at[pl.ds(row0, RPS)],
                                       zsem))
            pltpu.sync_copy(ones_hbm, ones_v)
        for d in zd:
            d.wait()
        plsc.subcore_barrier()

        for d in idescs[0]:
            d.wait()
        if NSB > 1:
            idescs[1] = stage_idx(1)
        gd = {0: pltpu.async_copy(h_hbm.at[srcb[0].at[0]], rows[0], sems[0])}
        for k in range(NCHUNK):
            b = k % 2
            sb, j = divmod(k, S)
            if k + 1 < NCHUNK:
                sbn, jn = divmod(k + 1, S)
                if jn == 0:
                    for d in idescs[sbn]:
                        d.wait()
                if jn == 1 and sbn + 1 < NSB:
                    idescs[sbn + 1] = stage_idx(sbn + 1)
                gd[k + 1] = pltpu.async_copy(
                    h_hbm.at[srcb[sbn % 2].at[jn]], rows[1 - b], sems[1 - b])
            gd[k].wait()
            pltpu.sync_copy(rows[b], agg_sh.at[dstb[sb % 2].at[j]], add=True)
            if with_deg:
                pltpu.sync_copy(ones_v, deg_sh.at[dstb[sb % 2].at[j]],
                                add=True)
        plsc.subcore_barrier()

        pltpu.sync_copy(agg_sh.at[pl.ds(row0, RPS)],
                        agg_out.at[c, pl.ds(row0, RPS)])
        if with_deg:
            pltpu.sync_copy(deg_sh.at[pl.ds(row0, RPS)],
                            deg_out.at[c, pl.ds(row0, RPS)])

    return body


_sc_agg_deg = _sc_agg(True)
_sc_agg_nodeg = _sc_agg(False)


def _tc_input(x, w_t, b):
    def body(x_ref, w_ref, b_ref, o_ref):
        acc = x_ref[:, 0:1] * w_ref[0:1, :] + x_ref[:, 1:2] * w_ref[1:2, :]
        o_ref[:] = jnp.maximum(acc + b_ref[:], 0.0)

    return pl.pallas_call(
        body,
        grid=(N // TB,),
        in_specs=[
            pl.BlockSpec((TB, 2), lambda i: (i, 0)),
            pl.BlockSpec((2, D), lambda i: (0, 0)),
            pl.BlockSpec((1, D), lambda i: (0, 0)),
        ],
        out_specs=pl.BlockSpec((TB, D), lambda i: (i, 0)),
        out_shape=jax.ShapeDtypeStruct((NPAD, D), jnp.float32),
    )(x, w_t, b)


def _tc_update(agg2, deg_t, h, wl_t, bl, wr_t, out_rows):
    def body(a_ref, d_ref, h_ref, wl_ref, bl_ref, wr_ref, o_ref):
        d = d_ref[:, 0] + d_ref[:, 1]
        mean = (a_ref[0] + a_ref[1]) / jnp.maximum(d, 1.0)[:, None]
        acc = jnp.dot(mean, wl_ref[:], preferred_element_type=jnp.float32)
        acc += jnp.dot(h_ref[:], wr_ref[:], preferred_element_type=jnp.float32)
        o_ref[:] = jnp.maximum(acc + bl_ref[:], 0.0)

    return pl.pallas_call(
        body,
        grid=(N // TB,),
        in_specs=[
            pl.BlockSpec((2, TB, D), lambda i: (0, i, 0)),
            pl.BlockSpec((TB, 2), lambda i: (i, 0)),
            pl.BlockSpec((TB, D), lambda i: (i, 0)),
            pl.BlockSpec((D, D), lambda i: (0, 0)),
            pl.BlockSpec((1, D), lambda i: (0, 0)),
            pl.BlockSpec((D, D), lambda i: (0, 0)),
        ],
        out_specs=pl.BlockSpec((TB, D), lambda i: (i, 0)),
        out_shape=jax.ShapeDtypeStruct((out_rows, D), jnp.float32),
    )(agg2, deg_t, h, wl_t, bl, wr_t)


def kernel(x, edge_index, W_in, b_in, Wl0, bl0, Wr0, Wl1, bl1, Wr1, Wl2, bl2, Wr2):
    src = edge_index[0].astype(jnp.int32).reshape(NW, NSB, S, C)
    dst = edge_index[1].astype(jnp.int32).reshape(NW, NSB, S, C)
    zeros_r = jnp.zeros((RPS, D), jnp.float32)
    zeros_d = jnp.zeros((RPS,), jnp.float32)
    ones_c = jnp.ones((C,), jnp.float32)

    h0 = _tc_input(x, W_in.T, b_in.reshape(1, D))

    agg2, deg2 = _sc_agg_deg(h0, src, dst, zeros_r, zeros_d, ones_c)
    deg_t = deg2.T
    h1 = _tc_update(agg2, deg_t, h0, Wl0.T, bl0.reshape(1, D), Wr0.T, NPAD)

    (agg2,) = _sc_agg_nodeg(h1, src, dst, zeros_r, zeros_d, ones_c)
    h2 = _tc_update(agg2, deg_t, h1, Wl1.T, bl1.reshape(1, D), Wr1.T, NPAD)

    (agg2,) = _sc_agg_nodeg(h2, src, dst, zeros_r, zeros_d, ones_c)
    h3 = _tc_update(agg2, deg_t, h2, Wl2.T, bl2.reshape(1, D), Wr2.T, N)

    return h3


# speedup vs baseline: 1.0253x; 1.0253x over previous
import functools

import jax
import jax.numpy as jnp
from jax import lax
from jax.experimental import pallas as pl
from jax.experimental.pallas import tpu as pltpu
from jax.experimental.pallas import tpu_sc as plsc

N = 10000
NPAD = 10240
E = 320000
D = 128
NW = 32
EPW = E // NW
C = 100
NCHUNK = EPW // C
S = 25
NSB = NCHUNK // S
RPS = NPAD // 16
TB = 2000


def _sc_agg(with_deg):
    out_type = [jax.ShapeDtypeStruct((2, NPAD, D), jnp.float32)]
    if with_deg:
        out_type.append(jax.ShapeDtypeStruct((2, NPAD), jnp.float32))

    mesh = plsc.VectorSubcoreMesh(core_axis_name="c", subcore_axis_name="s")
    scratch = [
        pltpu.VMEM_SHARED((NPAD, D), jnp.float32),
        pltpu.VMEM_SHARED((NPAD,), jnp.float32),
        pltpu.VMEM((S, C), jnp.int32),
        pltpu.VMEM((S, C), jnp.int32),
        pltpu.VMEM((S, C), jnp.int32),
        pltpu.VMEM((S, C), jnp.int32),
        pltpu.VMEM((C, D), jnp.float32),
        pltpu.VMEM((C, D), jnp.float32),
        pltpu.VMEM((C,), jnp.float32),
        pltpu.SemaphoreType.DMA,
        pltpu.SemaphoreType.DMA,
        pltpu.SemaphoreType.DMA,
        pltpu.SemaphoreType.DMA,
    ]

    @functools.partial(pl.kernel, out_type=out_type, mesh=mesh,
                       scratch_types=scratch)
    def body(h_hbm, src_hbm, dst_hbm, zero_hbm, zerod_hbm, ones_hbm, *rest):
        if with_deg:
            agg_out, deg_out = rest[0], rest[1]
            rest = rest[2:]
        else:
            agg_out = rest[0]
            rest = rest[1:]
        (agg_sh, deg_sh, src0, src1, dst0, dst1, rows0, rows1, ones_v,
         sem0, sem1, isem, zsem) = rest

        c = lax.axis_index("c")
        s = lax.axis_index("s")
        wid = c * 16 + s

        srcb = (src0, src1)
        dstb = (dst0, dst1)
        rows = (rows0, rows1)
        sems = (sem0, sem1)

        def stage_idx(sb):
            bb = sb % 2
            return (pltpu.async_copy(src_hbm.at[wid, sb], srcb[bb], isem),
                    pltpu.async_copy(dst_hbm.at[wid, sb], dstb[bb], isem))

        idescs = {0: stage_idx(0)}
        row0 = s * RPS
        zd = [pltpu.async_copy(zero_hbm, agg_sh.at[pl.ds(row0, RPS)], zsem)]
        if with_deg:
            zd.append(pltpu.async_copy(zerod_hbm, deg_sh.at[pl.ds(row0, RPS)],
                                       zsem))
            pltpu.sync_copy(ones_hbm, ones_v)
        for d in zd:
            d.wait()
        plsc.subcore_barrier()

        for d in idescs[0]:
            d.wait()
        if NSB > 1:
            idescs[1] = stage_idx(1)
        gd = {0: pltpu.async_copy(h_hbm.at[srcb[0].at[0]], rows[0], sems[0])}
        for k in range(NCHUNK):
            b = k % 2
            sb, j = divmod(k, S)
            if k + 1 < NCHUNK:
                sbn, jn = divmod(k + 1, S)
                if jn == 0:
                    for d in idescs[sbn]:
                        d.wait()
                if jn == 1 and sbn + 1 < NSB:
                    idescs[sbn + 1] = stage_idx(sbn + 1)
                gd[k + 1] = pltpu.async_copy(
                    h_hbm.at[srcb[sbn % 2].at[jn]], rows[1 - b], sems[1 - b])
            gd[k].wait()
            pltpu.sync_copy(rows[b], agg_sh.at[dstb[sb % 2].at[j]], add=True)
            if with_deg:
                pltpu.sync_copy(ones_v, deg_sh.at[dstb[sb % 2].at[j]],
                                add=True)
        plsc.subcore_barrier()

        od = [pltpu.async_copy(agg_sh.at[pl.ds(row0, RPS)],
                               agg_out.at[c, pl.ds(row0, RPS)], zsem)]
        if with_deg:
            od.append(pltpu.async_copy(deg_sh.at[pl.ds(row0, RPS)],
                                       deg_out.at[c, pl.ds(row0, RPS)], zsem))
        for d in od:
            d.wait()

    return body


_sc_agg_deg = _sc_agg(True)
_sc_agg_nodeg = _sc_agg(False)


def _tc_input(x, w_t, b):
    def body(x_ref, w_ref, b_ref, o_ref):
        acc = x_ref[:, 0:1] * w_ref[0:1, :] + x_ref[:, 1:2] * w_ref[1:2, :]
        o_ref[:] = jnp.maximum(acc + b_ref[:], 0.0)

    return pl.pallas_call(
        body,
        grid=(N // TB,),
        in_specs=[
            pl.BlockSpec((TB, 2), lambda i: (i, 0)),
            pl.BlockSpec((2, D), lambda i: (0, 0)),
            pl.BlockSpec((1, D), lambda i: (0, 0)),
        ],
        out_specs=pl.BlockSpec((TB, D), lambda i: (i, 0)),
        out_shape=jax.ShapeDtypeStruct((NPAD, D), jnp.float32),
    )(x, w_t, b)


def _tc_update(agg2, deg_t, h, wl_t, bl, wr_t, out_rows):
    def body(a_ref, d_ref, h_ref, wl_ref, bl_ref, wr_ref, o_ref):
        d = d_ref[:, 0] + d_ref[:, 1]
        mean = (a_ref[0] + a_ref[1]) / jnp.maximum(d, 1.0)[:, None]
        acc = jnp.dot(mean, wl_ref[:], preferred_element_type=jnp.float32)
        acc += jnp.dot(h_ref[:], wr_ref[:], preferred_element_type=jnp.float32)
        o_ref[:] = jnp.maximum(acc + bl_ref[:], 0.0)

    return pl.pallas_call(
        body,
        grid=(N // TB,),
        in_specs=[
            pl.BlockSpec((2, TB, D), lambda i: (0, i, 0)),
            pl.BlockSpec((TB, 2), lambda i: (i, 0)),
            pl.BlockSpec((TB, D), lambda i: (i, 0)),
            pl.BlockSpec((D, D), lambda i: (0, 0)),
            pl.BlockSpec((1, D), lambda i: (0, 0)),
            pl.BlockSpec((D, D), lambda i: (0, 0)),
        ],
        out_specs=pl.BlockSpec((TB, D), lambda i: (i, 0)),
        out_shape=jax.ShapeDtypeStruct((out_rows, D), jnp.float32),
    )(agg2, deg_t, h, wl_t, bl, wr_t)


def kernel(x, edge_index, W_in, b_in, Wl0, bl0, Wr0, Wl1, bl1, Wr1, Wl2, bl2, Wr2):
    src = edge_index[0].astype(jnp.int32).reshape(NW, NSB, S, C)
    dst = edge_index[1].astype(jnp.int32).reshape(NW, NSB, S, C)
    zeros_r = jnp.zeros((RPS, D), jnp.float32)
    zeros_d = jnp.zeros((RPS,), jnp.float32)
    ones_c = jnp.ones((C,), jnp.float32)

    h0 = _tc_input(x, W_in.T, b_in.reshape(1, D))

    agg2, deg2 = _sc_agg_deg(h0, src, dst, zeros_r, zeros_d, ones_c)
    deg_t = deg2.T
    h1 = _tc_update(agg2, deg_t, h0, Wl0.T, bl0.reshape(1, D), Wr0.T, NPAD)

    (agg2,) = _sc_agg_nodeg(h1, src, dst, zeros_r, zeros_d, ones_c)
    h2 = _tc_update(agg2, deg_t, h1, Wl1.T, bl1.reshape(1, D), Wr1.T, NPAD)

    (agg2,) = _sc_agg_nodeg(h2, src, dst, zeros_r, zeros_d, ones_c)
    h3 = _tc_update(agg2, deg_t, h2, Wl2.T, bl2.reshape(1, D), Wr2.T, N)

    return h3


# speedup vs baseline: 1.0297x; 1.0043x over previous
import functools

import jax
import jax.numpy as jnp
from jax import lax
from jax.experimental import pallas as pl
from jax.experimental.pallas import tpu as pltpu
from jax.experimental.pallas import tpu_sc as plsc

N = 10000
NPAD = 10240
E = 320000
D = 128
NW = 32
EPW = E // NW
C = 100
NCHUNK = EPW // C
S = 25
NSB = NCHUNK // S
RPS = NPAD // 16
TB = 2000


def _sc_agg(with_deg):
    out_type = [jax.ShapeDtypeStruct((2, NPAD, D), jnp.float32)]
    if with_deg:
        out_type.append(jax.ShapeDtypeStruct((2, NPAD), jnp.float32))

    mesh = plsc.VectorSubcoreMesh(core_axis_name="c", subcore_axis_name="s")
    scratch = [
        pltpu.VMEM_SHARED((NPAD, D), jnp.float32),
        pltpu.VMEM_SHARED((NPAD,), jnp.float32),
        pltpu.VMEM((S, C), jnp.int32),
        pltpu.VMEM((S, C), jnp.int32),
        pltpu.VMEM((S, C), jnp.int32),
        pltpu.VMEM((S, C), jnp.int32),
        pltpu.VMEM((C, D), jnp.float32),
        pltpu.VMEM((C, D), jnp.float32),
        pltpu.VMEM((C,), jnp.float32),
        pltpu.SemaphoreType.DMA,
        pltpu.SemaphoreType.DMA,
        pltpu.SemaphoreType.DMA,
        pltpu.SemaphoreType.DMA,
        pltpu.SemaphoreType.DMA,
        pltpu.SemaphoreType.DMA,
    ]

    @functools.partial(pl.kernel, out_type=out_type, mesh=mesh,
                       scratch_types=scratch)
    def body(h_hbm, src_hbm, dst_hbm, zero_hbm, zerod_hbm, ones_hbm, *rest):
        if with_deg:
            agg_out, deg_out = rest[0], rest[1]
            rest = rest[2:]
        else:
            agg_out = rest[0]
            rest = rest[1:]
        (agg_sh, deg_sh, src0, src1, dst0, dst1, rows0, rows1, ones_v,
         sem0, sem1, isem, zsem, ssem0, ssem1) = rest
        ssems = (ssem0, ssem1)

        c = lax.axis_index("c")
        s = lax.axis_index("s")
        wid = c * 16 + s

        srcb = (src0, src1)
        dstb = (dst0, dst1)
        rows = (rows0, rows1)
        sems = (sem0, sem1)

        def stage_idx(sb):
            bb = sb % 2
            return (pltpu.async_copy(src_hbm.at[wid, sb], srcb[bb], isem),
                    pltpu.async_copy(dst_hbm.at[wid, sb], dstb[bb], isem))

        idescs = {0: stage_idx(0)}
        row0 = s * RPS
        zd = [pltpu.async_copy(zero_hbm, agg_sh.at[pl.ds(row0, RPS)], zsem)]
        if with_deg:
            zd.append(pltpu.async_copy(zerod_hbm, deg_sh.at[pl.ds(row0, RPS)],
                                       zsem))
            pltpu.sync_copy(ones_hbm, ones_v)
        for d in zd:
            d.wait()
        plsc.subcore_barrier()

        for d in idescs[0]:
            d.wait()
        if NSB > 1:
            idescs[1] = stage_idx(1)
        gd = {0: pltpu.async_copy(h_hbm.at[srcb[0].at[0]], rows[0], sems[0])}
        sd = {}
        for k in range(NCHUNK):
            b = k % 2
            sb, j = divmod(k, S)
            if k + 1 < NCHUNK:
                sbn, jn = divmod(k + 1, S)
                if jn == 0:
                    for d in idescs[sbn]:
                        d.wait()
                if k >= 1:
                    for d in sd.pop(k - 1):
                        d.wait()
                if jn == 1 and sbn + 1 < NSB:
                    idescs[sbn + 1] = stage_idx(sbn + 1)
                gd[k + 1] = pltpu.async_copy(
                    h_hbm.at[srcb[sbn % 2].at[jn]], rows[1 - b], sems[1 - b])
            gd[k].wait()
            scat = [pltpu.async_copy(rows[b], agg_sh.at[dstb[sb % 2].at[j]],
                                     ssems[b], add=True)]
            if with_deg:
                scat.append(pltpu.async_copy(
                    ones_v, deg_sh.at[dstb[sb % 2].at[j]], ssems[b], add=True))
            sd[k] = scat
        for k in sorted(sd):
            for d in sd[k]:
                d.wait()
        plsc.subcore_barrier()

        od = [pltpu.async_copy(agg_sh.at[pl.ds(row0, RPS)],
                               agg_out.at[c, pl.ds(row0, RPS)], zsem)]
        if with_deg:
            od.append(pltpu.async_copy(deg_sh.at[pl.ds(row0, RPS)],
                                       deg_out.at[c, pl.ds(row0, RPS)], zsem))
        for d in od:
            d.wait()

    return body


_sc_agg_deg = _sc_agg(True)
_sc_agg_nodeg = _sc_agg(False)


def _tc_input(x, w_t, b):
    def body(x_ref, w_ref, b_ref, o_ref):
        acc = x_ref[:, 0:1] * w_ref[0:1, :] + x_ref[:, 1:2] * w_ref[1:2, :]
        o_ref[:] = jnp.maximum(acc + b_ref[:], 0.0)

    return pl.pallas_call(
        body,
        grid=(N // TB,),
        in_specs=[
            pl.BlockSpec((TB, 2), lambda i: (i, 0)),
            pl.BlockSpec((2, D), lambda i: (0, 0)),
            pl.BlockSpec((1, D), lambda i: (0, 0)),
        ],
        out_specs=pl.BlockSpec((TB, D), lambda i: (i, 0)),
        out_shape=jax.ShapeDtypeStruct((NPAD, D), jnp.float32),
    )(x, w_t, b)


def _tc_update(agg2, deg_t, h, wl_t, bl, wr_t, out_rows):
    def body(a_ref, d_ref, h_ref, wl_ref, bl_ref, wr_ref, o_ref):
        d = d_ref[:, 0] + d_ref[:, 1]
        mean = (a_ref[0] + a_ref[1]) / jnp.maximum(d, 1.0)[:, None]
        acc = jnp.dot(mean, wl_ref[:], preferred_element_type=jnp.float32)
        acc += jnp.dot(h_ref[:], wr_ref[:], preferred_element_type=jnp.float32)
        o_ref[:] = jnp.maximum(acc + bl_ref[:], 0.0)

    return pl.pallas_call(
        body,
        grid=(N // TB,),
        in_specs=[
            pl.BlockSpec((2, TB, D), lambda i: (0, i, 0)),
            pl.BlockSpec((TB, 2), lambda i: (i, 0)),
            pl.BlockSpec((TB, D), lambda i: (i, 0)),
            pl.BlockSpec((D, D), lambda i: (0, 0)),
            pl.BlockSpec((1, D), lambda i: (0, 0)),
            pl.BlockSpec((D, D), lambda i: (0, 0)),
        ],
        out_specs=pl.BlockSpec((TB, D), lambda i: (i, 0)),
        out_shape=jax.ShapeDtypeStruct((out_rows, D), jnp.float32),
    )(agg2, deg_t, h, wl_t, bl, wr_t)


def kernel(x, edge_index, W_in, b_in, Wl0, bl0, Wr0, Wl1, bl1, Wr1, Wl2, bl2, Wr2):
    src = edge_index[0].astype(jnp.int32).reshape(NW, NSB, S, C)
    dst = edge_index[1].astype(jnp.int32).reshape(NW, NSB, S, C)
    zeros_r = jnp.zeros((RPS, D), jnp.float32)
    zeros_d = jnp.zeros((RPS,), jnp.float32)
    ones_c = jnp.ones((C,), jnp.float32)

    h0 = _tc_input(x, W_in.T, b_in.reshape(1, D))

    agg2, deg2 = _sc_agg_deg(h0, src, dst, zeros_r, zeros_d, ones_c)
    deg_t = deg2.T
    h1 = _tc_update(agg2, deg_t, h0, Wl0.T, bl0.reshape(1, D), Wr0.T, NPAD)

    (agg2,) = _sc_agg_nodeg(h1, src, dst, zeros_r, zeros_d, ones_c)
    h2 = _tc_update(agg2, deg_t, h1, Wl1.T, bl1.reshape(1, D), Wr1.T, NPAD)

    (agg2,) = _sc_agg_nodeg(h2, src, dst, zeros_r, zeros_d, ones_c)
    h3 = _tc_update(agg2, deg_t, h2, Wl2.T, bl2.reshape(1, D), Wr2.T, N)

    return h3
